# baseline (speedup 1.0000x reference)
import functools

import jax
import jax.numpy as jnp
from jax import lax
from jax.experimental import pallas as pl
from jax.experimental.pallas import tpu as pltpu
from jax.experimental.pallas import tpu_sc as plsc

B = 4
SEQ = 2048
HIDDEN = 128
N_ROWS = B * SEQ
N_WORKERS = 32
ROWS_PER_WORKER = N_ROWS // N_WORKERS


def _sc_gather(tok_table, flat_ids):
    mesh = plsc.VectorSubcoreMesh(core_axis_name="c", subcore_axis_name="s")

    half = ROWS_PER_WORKER // 2

    @functools.partial(
        pl.kernel, mesh=mesh,
        out_type=jax.ShapeDtypeStruct((N_ROWS, HIDDEN), jnp.float32),
        scratch_types=[
            pltpu.VMEM((half,), jnp.int32),
            pltpu.VMEM((half,), jnp.int32),
            pltpu.VMEM((half, HIDDEN), jnp.float32),
            pltpu.VMEM((half, HIDDEN), jnp.float32),
            pltpu.SemaphoreType.DMA,
            pltpu.SemaphoreType.DMA,
            pltpu.SemaphoreType.DMA,
        ],
    )
    def gather_kernel(tab_hbm, idx_hbm, out_hbm,
                      idx0_v, idx1_v, rows0_v, rows1_v, sem0, sem1, sem_s):
        wid = lax.axis_index("s") * 2 + lax.axis_index("c")
        base = wid * ROWS_PER_WORKER
        pltpu.sync_copy(idx_hbm.at[pl.ds(base, half)], idx0_v)
        pltpu.sync_copy(idx_hbm.at[pl.ds(base + half, half)], idx1_v)
        g0 = pltpu.async_copy(tab_hbm.at[idx0_v], rows0_v, sem0)
        g1 = pltpu.async_copy(tab_hbm.at[idx1_v], rows1_v, sem1)
        g0.wait()
        s0 = pltpu.async_copy(rows0_v, out_hbm.at[pl.ds(base, half)], sem_s)
        g1.wait()
        s1 = pltpu.async_copy(rows1_v, out_hbm.at[pl.ds(base + half, half)],
                              sem_s)
        s0.wait()
        s1.wait()

    return gather_kernel(tok_table, flat_ids)


def _tc_dense(tok_emb, seg_ids3, pos_table, seg_table, gamma2, beta2):
    def body(x_ref, sid_ref, pos_ref, segtab_ref, gamma_ref, beta_ref, o_ref):
        x = x_ref[0] + pos_ref[...]
        sid = sid_ref[0, 0]
        seg = jnp.where((sid[:, None]) == 0,
                        segtab_ref[0:1, :], segtab_ref[1:2, :])
        x = x + seg
        mu = jnp.mean(x, axis=-1, keepdims=True)
        var = jnp.mean((x - mu) ** 2, axis=-1, keepdims=True)
        xhat = (x - mu) * jax.lax.rsqrt(var + 1e-5)
        o_ref[0] = xhat * gamma_ref[...] + beta_ref[...]

    return pl.pallas_call(
        body,
        grid=(B,),
        in_specs=[
            pl.BlockSpec((1, SEQ, HIDDEN), lambda b: (b, 0, 0)),
            pl.BlockSpec((1, 1, SEQ), lambda b: (b, 0, 0)),
            pl.BlockSpec((SEQ, HIDDEN), lambda b: (0, 0)),
            pl.BlockSpec((2, HIDDEN), lambda b: (0, 0)),
            pl.BlockSpec((1, HIDDEN), lambda b: (0, 0)),
            pl.BlockSpec((1, HIDDEN), lambda b: (0, 0)),
        ],
        out_specs=pl.BlockSpec((1, SEQ, HIDDEN), lambda b: (b, 0, 0)),
        out_shape=jax.ShapeDtypeStruct((B, SEQ, HIDDEN), jnp.float32),
    )(tok_emb, seg_ids3, pos_table, seg_table, gamma2, beta2)


def kernel(token_ids, seg_ids, tok_table, pos_table, seg_table, gamma, beta):
    flat_ids = token_ids.astype(jnp.int32).reshape(N_ROWS)
    tok_emb = _sc_gather(tok_table, flat_ids).reshape(B, SEQ, HIDDEN)
    seg_ids3 = seg_ids.astype(jnp.int32).reshape(B, 1, SEQ)
    gamma2 = gamma.reshape(1, HIDDEN)
    beta2 = beta.reshape(1, HIDDEN)
    return _tc_dense(tok_emb, seg_ids3, pos_table, seg_table, gamma2, beta2)
